# baseline (speedup 1.0000x reference)
import functools

import jax
import jax.numpy as jnp
from jax import lax
from jax.experimental import pallas as pl
from jax.experimental.pallas import tpu as pltpu
from jax.experimental.pallas import tpu_sc as plsc

_DATA_DIM = 26
_DEPTH = 1000
_BATCH = 1024
_COLS = _DATA_DIM * _DEPTH
_NW = 32
_ROWS_PER_W = _BATCH // _NW
_BANDS_PER_W = _ROWS_PER_W // 8
_CHUNK = 6400
_NCHUNK = 4
_TAIL0 = _NCHUNK * _CHUNK
_TAIL = _COLS - _TAIL0


def _scatter_chunk(buf, xbuf, band, c0, clen, val):
    lanes = jax.lax.iota(jnp.int32, 16)
    row_local = lanes // 8
    dd = lanes - 8 * row_local
    dlo = c0 // _DEPTH
    ndim = (c0 + clen - 1) // _DEPTH - dlo + 1
    d = dlo + dd
    for g in range(4):
        xrow = band * 8 + 2 * g + row_local
        xv = plsc.load_gather(xbuf, [xrow, d])
        col = d * _DEPTH + xv - c0
        valid = (col >= 0) & (col < clen) & (dd < ndim)
        plsc.store_scatter(
            buf, [2 * g + row_local, col], val, mask=valid
        )


def _scatter_tail(buf, xbuf, band, val):
    lanes = jax.lax.iota(jnp.int32, 16)
    row_local = lanes - 8 * (lanes // 8)
    half = lanes // 8
    xrow = band * 8 + row_local
    d = jnp.full((16,), _DATA_DIM - 1, jnp.int32)
    xv = plsc.load_gather(xbuf, [xrow, d])
    col = xv - (_TAIL0 - (_DATA_DIM - 1) * _DEPTH)
    valid = (col >= 0) & (col < _TAIL) & (half < 1)
    plsc.store_scatter(buf, [row_local, col], val, mask=valid)


def _sc_body(x_hbm, out_hbm, xbuf, big0, big1, tail0, tail1, sems):
    nc = 2
    wid = lax.axis_index("s") * nc + lax.axis_index("c")
    row_base = pl.multiple_of(wid * _ROWS_PER_W, 8)

    pltpu.sync_copy(x_hbm.at[pl.ds(row_base, _ROWS_PER_W), :], xbuf)

    zeros = jnp.zeros((16,), jnp.float32)
    ones = jnp.ones((16,), jnp.float32)

    bigs = (big0, big1)
    tails = (tail0, tail1)

    for buf in bigs:
        for r in range(8):
            def zbody(i, _, buf=buf, r=r):
                buf[r, pl.ds(i * 16, 16)] = zeros
                return 0

            lax.fori_loop(0, _CHUNK // 16, zbody, 0)
    for buf in tails:
        for r in range(8):
            def ztail(i, _, buf=buf, r=r):
                buf[r, pl.ds(i * 16, 16)] = zeros
                return 0

            lax.fori_loop(0, _TAIL // 16, ztail, 0)

    def big_dst(band, ci):
        ro = pl.multiple_of(row_base + band * 8, 8)
        return out_hbm.at[pl.ds(ro, 8), pl.ds(ci * _CHUNK, _CHUNK)]

    def tail_dst(band):
        ro = pl.multiple_of(row_base + band * 8, 8)
        return out_hbm.at[pl.ds(ro, 8), pl.ds(_TAIL0, _TAIL)]

    big_hist = []
    tail_hist = []
    for band in range(_BANDS_PER_W):
        for ci in range(_NCHUNK):
            seq = len(big_hist)
            slot = seq % 2
            if seq >= 2:
                pband, pci = big_hist[seq - 2]
                pltpu.make_async_copy(
                    bigs[slot], big_dst(pband, pci), sems.at[slot]
                ).wait()
                _scatter_chunk(
                    bigs[slot], xbuf, pband, pci * _CHUNK, _CHUNK, zeros
                )
            _scatter_chunk(bigs[slot], xbuf, band, ci * _CHUNK, _CHUNK, ones)
            pltpu.make_async_copy(
                bigs[slot], big_dst(band, ci), sems.at[slot]
            ).start()
            big_hist.append((band, ci))

        tseq = len(tail_hist)
        tslot = tseq % 2
        if tseq >= 2:
            pband = tail_hist[tseq - 2]
            pltpu.make_async_copy(
                tails[tslot], tail_dst(pband), sems.at[2 + tslot]
            ).wait()
            _scatter_tail(tails[tslot], xbuf, pband, zeros)
        _scatter_tail(tails[tslot], xbuf, band, ones)
        pltpu.make_async_copy(
            tails[tslot], tail_dst(band), sems.at[2 + tslot]
        ).start()
        tail_hist.append(band)

    for off in (2, 1):
        seq = len(big_hist) - off
        pband, pci = big_hist[seq]
        pltpu.make_async_copy(
            bigs[seq % 2], big_dst(pband, pci), sems.at[seq % 2]
        ).wait()
    for off in (2, 1):
        tseq = len(tail_hist) - off
        pband = tail_hist[tseq]
        pltpu.make_async_copy(
            tails[tseq % 2], tail_dst(pband), sems.at[2 + tseq % 2]
        ).wait()


def kernel(x):
    mesh = plsc.VectorSubcoreMesh(core_axis_name="c", subcore_axis_name="s")
    k = functools.partial(
        pl.kernel,
        mesh=mesh,
        compiler_params=pltpu.CompilerParams(needs_layout_passes=False, use_tc_tiling_on_sc=True),
        out_type=jax.ShapeDtypeStruct((_BATCH, _COLS), jnp.float32),
        scratch_types=[
            pltpu.VMEM((_ROWS_PER_W, _DATA_DIM), jnp.int32),
            pltpu.VMEM((8, _CHUNK), jnp.float32),
            pltpu.VMEM((8, _CHUNK), jnp.float32),
            pltpu.VMEM((8, _TAIL), jnp.float32),
            pltpu.VMEM((8, _TAIL), jnp.float32),
            pltpu.SemaphoreType.DMA((4,)),
        ],
    )(_sc_body)
    return k(x)


# speedup vs baseline: 5.0161x; 5.0161x over previous
import jax
import jax.numpy as jnp
from jax import lax
from jax.experimental import pallas as pl

_DATA_DIM = 26
_DEPTH = 1000
_BATCH = 1024


def _body(xt_ref, o_ref):
    v = lax.broadcasted_iota(jnp.int32, (_DEPTH, _BATCH), 0)
    o_ref[...] = (v == xt_ref[0]).astype(jnp.float32)


def kernel(x):
    xt = x.T.reshape(_DATA_DIM, 1, _BATCH)
    ot = pl.pallas_call(
        _body,
        grid=(_DATA_DIM,),
        in_specs=[pl.BlockSpec((1, 1, _BATCH), lambda i: (i, 0, 0))],
        out_specs=pl.BlockSpec((_DEPTH, _BATCH), lambda i: (i, 0)),
        out_shape=jax.ShapeDtypeStruct((_DATA_DIM * _DEPTH, _BATCH), jnp.float32),
    )(xt)
    return ot.T
